# baseline (speedup 1.0000x reference)
import functools

import jax
import jax.numpy as jnp
from jax import lax
from jax.experimental import pallas as pl
from jax.experimental.pallas import tpu as pltpu
from jax.experimental.pallas import tpu_sc as plsc

N_CLUSTERS = 1024
EMBED_DIM = 256
DECAY = 0.99
EPS = 1e-05
BATCH = 16384

_BB = 1024


def _assign_body(xn_ref, xn2_ref, c_ref, c2_ref, idx_ref):
    xn = xn_ref[...]
    c = c_ref[...]
    mm = lax.dot_general(xn, c, (((1,), (1,)), ((), ())))
    d = (xn2_ref[...] + c2_ref[...]) - 2.0 * mm
    m = jnp.min(d, axis=1, keepdims=True)
    ii = lax.broadcasted_iota(jnp.int32, d.shape, 1)
    idx_ref[...] = jnp.min(jnp.where(d == m, ii, N_CLUSTERS), axis=1,
                           keepdims=True)


def _assign(xn, xn2, c2, centroids):
    return pl.pallas_call(
        _assign_body,
        grid=(BATCH // _BB,),
        in_specs=[
            pl.BlockSpec((_BB, EMBED_DIM), lambda i: (i, 0)),
            pl.BlockSpec((_BB, 1), lambda i: (i, 0)),
            pl.BlockSpec((N_CLUSTERS, EMBED_DIM), lambda i: (0, 0)),
            pl.BlockSpec((1, N_CLUSTERS), lambda i: (0, 0)),
        ],
        out_specs=pl.BlockSpec((_BB, 1), lambda i: (i, 0)),
        out_shape=jax.ShapeDtypeStruct((BATCH, 1), jnp.int32),
    )(xn, xn2, centroids, c2)


_NC, _NS, _L = 2, 16, 16
_NW = _NC * _NS
_RW = BATCH // _NW
_CH = 128
_NSUB = _RW // _CH

_sc_mesh = plsc.VectorSubcoreMesh(core_axis_name="c", subcore_axis_name="s")


@functools.partial(
    pl.kernel,
    mesh=_sc_mesh,
    out_type=jax.ShapeDtypeStruct((BATCH, EMBED_DIM), jnp.float32),
    scratch_types=[
        pltpu.VMEM((_NSUB, _CH), jnp.int32),
        pltpu.VMEM((_CH, EMBED_DIM), jnp.float32),
        pltpu.VMEM((_CH, EMBED_DIM), jnp.float32),
        pltpu.SemaphoreType.DMA,
        pltpu.SemaphoreType.DMA,
    ],
)
def _sc_gather(idx3_hbm, cent_hbm, quant_hbm, idx_v, buf0, buf1, sem0, sem1):
    cid = lax.axis_index("c")
    sid = lax.axis_index("s")
    wid = sid * _NC + cid
    base = wid * _RW
    pltpu.sync_copy(idx3_hbm.at[wid], idx_v)
    bufs = (buf0, buf1)
    sems = (sem0, sem1)
    gets = []
    for j in range(_NSUB):
        gets.append(pltpu.async_copy(cent_hbm.at[idx_v.at[j]], bufs[j % 2],
                                     sems[j % 2]))
        if j > 0:
            gets[j - 1].wait()
            pltpu.sync_copy(bufs[(j - 1) % 2],
                            quant_hbm.at[pl.ds(base + (j - 1) * _CH, _CH)])
    gets[_NSUB - 1].wait()
    pltpu.sync_copy(bufs[(_NSUB - 1) % 2],
                    quant_hbm.at[pl.ds(base + (_NSUB - 1) * _CH, _CH)])


def _update_body(idx_ref, x_ref, ema_sz_ref, ema_w_ref, out_ref,
                 dw_acc, cnt_acc):
    i = pl.program_id(0)
    ii = lax.broadcasted_iota(jnp.int32, (_BB, N_CLUSTERS), 1)
    oh = (ii == idx_ref[...]).astype(jnp.float32)
    ones_col = jnp.ones((_BB, 1), jnp.float32)
    dw_p = lax.dot_general(oh, x_ref[...], (((0,), (0,)), ((), ())))
    cnt_p = lax.dot_general(oh, ones_col, (((0,), (0,)), ((), ())))

    @pl.when(i == 0)
    def _():
        dw_acc[...] = dw_p
        cnt_acc[...] = cnt_p

    @pl.when(i > 0)
    def _():
        dw_acc[...] += dw_p
        cnt_acc[...] += cnt_p

    @pl.when(i == pl.num_programs(0) - 1)
    def _():
        ns = ema_sz_ref[...] * DECAY + (1.0 - DECAY) * cnt_acc[...]
        n = jnp.sum(ns)
        ns2 = (ns + EPS) / (n + N_CLUSTERS * EPS) * n
        new_ema_w = ema_w_ref[...] * DECAY + (1.0 - DECAY) * dw_acc[...]
        out_ref[...] = new_ema_w / ns2


def _update(idx2, x, ema_sz, ema_w):
    return pl.pallas_call(
        _update_body,
        grid=(BATCH // _BB,),
        in_specs=[
            pl.BlockSpec((_BB, 1), lambda i: (i, 0)),
            pl.BlockSpec((_BB, EMBED_DIM), lambda i: (i, 0)),
            pl.BlockSpec((N_CLUSTERS, 1), lambda i: (0, 0)),
            pl.BlockSpec((N_CLUSTERS, EMBED_DIM), lambda i: (0, 0)),
        ],
        out_specs=pl.BlockSpec((N_CLUSTERS, EMBED_DIM), lambda i: (0, 0)),
        out_shape=jax.ShapeDtypeStruct((N_CLUSTERS, EMBED_DIM), jnp.float32),
        scratch_shapes=[
            pltpu.VMEM((N_CLUSTERS, EMBED_DIM), jnp.float32),
            pltpu.VMEM((N_CLUSTERS, 1), jnp.float32),
        ],
    )(idx2, x, ema_sz, ema_w)


def kernel(X, centroids, ema_cluster_size, ema_w):
    norm = jnp.linalg.norm(X, ord=2, axis=1, keepdims=True)
    Xn = X / jnp.clip(norm, 1e-12, None)
    xn2 = jnp.sum(Xn ** 2, axis=1, keepdims=True)
    c2 = jnp.sum(centroids ** 2, axis=1)[None, :]

    idx2 = _assign(Xn, xn2, c2, centroids)
    idx3 = idx2.reshape(_NW, _NSUB, _CH)

    quant = _sc_gather(idx3, centroids)
    new_centroids = _update(idx2, X, ema_cluster_size.reshape(N_CLUSTERS, 1),
                            ema_w)
    return quant, idx2, new_centroids
